# Initial kernel scaffold; baseline (speedup 1.0000x reference)
import jax
import jax.numpy as jnp
from jax.experimental import pallas as pl


def kernel(x, edge_index, W1, b1, W2, b2):
    raise NotImplementedError("write your pallas kernel here")


# speedup vs baseline: 12.5000x; 12.5000x over previous
import functools

import jax
import jax.numpy as jnp
from jax import lax
from jax.experimental import pallas as pl
from jax.experimental.pallas import tpu as pltpu
from jax.experimental.pallas import tpu_sc as plsc

N = 10000
NP = 10240
E = 320000
D = 128
NC = 2
NS = 16
NW = NC * NS
EPW = E // NW
C = 80
K = EPW // C
RPT = NP // NS
ZF = RPT // C

_MESH = plsc.VectorSubcoreMesh(core_axis_name="c", subcore_axis_name="s")


def _fill_rows(buf, nrow, ncol, vec):
    def body(i, _):
        for j in range(ncol // 16):
            buf[i, pl.ds(j * 16, 16)] = vec
        return 0

    lax.fori_loop(0, nrow, body, 0)


def _cnt_body(dst_hbm, out_hbm, didx, ones_v, acc, sem):
    c = lax.axis_index("c")
    s = lax.axis_index("s")
    w = c * NS + s
    _fill_rows(ones_v, C, D, jnp.zeros((16,), jnp.float32))
    for k in range(ZF):
        pltpu.sync_copy(ones_v, acc.at[pl.ds(s * RPT + k * C, C)])
    plsc.subcore_barrier()
    _fill_rows(ones_v, C, D, jnp.ones((16,), jnp.float32))

    def chunk(k, _):
        base = w * EPW + k * C
        pltpu.sync_copy(dst_hbm.at[pl.ds(base, C)], didx)
        pltpu.sync_copy(ones_v, acc.at[didx], add=True)
        return 0

    lax.fori_loop(0, K, chunk, 0)
    plsc.subcore_barrier()
    for k in range(ZF):
        pltpu.sync_copy(acc.at[pl.ds(s * RPT + k * C, C)],
                        out_hbm.at[pl.ds(c * NP + s * RPT + k * C, C)])


_cnt_call = pl.kernel(
    _cnt_body,
    out_type=jax.ShapeDtypeStruct((2 * NP, D), jnp.float32),
    mesh=_MESH,
    scratch_types=[
        pltpu.VMEM((C,), jnp.int32),
        pltpu.VMEM((C, D), jnp.float32),
        pltpu.VMEM_SHARED((NP, D), jnp.float32),
        pltpu.SemaphoreType.DMA,
    ],
)


def _agg_body(hs_hbm, src_hbm, dst_hbm, out_hbm, sidx, didx, rows, acc, sem):
    c = lax.axis_index("c")
    s = lax.axis_index("s")
    w = c * NS + s
    _fill_rows(rows, C, D, jnp.zeros((16,), jnp.float32))
    for k in range(ZF):
        pltpu.sync_copy(rows, acc.at[pl.ds(s * RPT + k * C, C)])
    plsc.subcore_barrier()

    def chunk(k, _):
        base = w * EPW + k * C
        pltpu.sync_copy(src_hbm.at[pl.ds(base, C)], sidx)
        pltpu.sync_copy(dst_hbm.at[pl.ds(base, C)], didx)
        pltpu.async_copy(hs_hbm.at[sidx], rows, sem).wait()
        pltpu.sync_copy(rows, acc.at[didx], add=True)
        return 0

    lax.fori_loop(0, K, chunk, 0)
    plsc.subcore_barrier()
    for k in range(ZF):
        pltpu.sync_copy(acc.at[pl.ds(s * RPT + k * C, C)],
                        out_hbm.at[pl.ds(c * NP + s * RPT + k * C, C)])


_agg_call = pl.kernel(
    _agg_body,
    out_type=jax.ShapeDtypeStruct((2 * NP, D), jnp.float32),
    mesh=_MESH,
    scratch_types=[
        pltpu.VMEM((C,), jnp.int32),
        pltpu.VMEM((C,), jnp.int32),
        pltpu.VMEM((C, D), jnp.float32),
        pltpu.VMEM_SHARED((NP, D), jnp.float32),
        pltpu.SemaphoreType.DMA,
    ],
)

BR = 1024


def _pre_body(cnt_ref, x_ref, w_ref, dinv_ref, hs_ref):
    deg = cnt_ref[0] + cnt_ref[1] + 1.0
    dinvb = lax.rsqrt(jnp.maximum(deg, 1e-12))
    h = jnp.dot(x_ref[...], w_ref[...], preferred_element_type=jnp.float32)
    dinv_ref[...] = dinvb
    hs_ref[...] = h * dinvb


_pre_call = pl.pallas_call(
    _pre_body,
    grid=(pl.cdiv(N, BR),),
    in_specs=[
        pl.BlockSpec((2, BR, D), lambda i: (0, i, 0)),
        pl.BlockSpec((BR, D), lambda i: (i, 0)),
        pl.BlockSpec((D, D), lambda i: (0, 0)),
    ],
    out_specs=[
        pl.BlockSpec((BR, D), lambda i: (i, 0)),
        pl.BlockSpec((BR, D), lambda i: (i, 0)),
    ],
    out_shape=[
        jax.ShapeDtypeStruct((N, D), jnp.float32),
        jax.ShapeDtypeStruct((N, D), jnp.float32),
    ],
)


def _mid_body(p_ref, hs1_ref, dinv_ref, b1_ref, w2_ref, hs2_ref):
    agg = p_ref[0] + p_ref[1] + hs1_ref[...]
    t = agg * dinv_ref[...] + b1_ref[...]
    t = jnp.maximum(t, 0.0)
    h2 = jnp.dot(t, w2_ref[...], preferred_element_type=jnp.float32)
    hs2_ref[...] = h2 * dinv_ref[...]


_mid_call = pl.pallas_call(
    _mid_body,
    grid=(pl.cdiv(N, BR),),
    in_specs=[
        pl.BlockSpec((2, BR, D), lambda i: (0, i, 0)),
        pl.BlockSpec((BR, D), lambda i: (i, 0)),
        pl.BlockSpec((BR, D), lambda i: (i, 0)),
        pl.BlockSpec((1, D), lambda i: (0, 0)),
        pl.BlockSpec((D, D), lambda i: (0, 0)),
    ],
    out_specs=pl.BlockSpec((BR, D), lambda i: (i, 0)),
    out_shape=jax.ShapeDtypeStruct((N, D), jnp.float32),
)


def _fin_body(q_ref, hs2_ref, dinv_ref, b2_ref, out_ref):
    agg = q_ref[0] + q_ref[1] + hs2_ref[...]
    out_ref[...] = agg * dinv_ref[...] + b2_ref[...]


_fin_call = pl.pallas_call(
    _fin_body,
    grid=(pl.cdiv(N, BR),),
    in_specs=[
        pl.BlockSpec((2, BR, D), lambda i: (0, i, 0)),
        pl.BlockSpec((BR, D), lambda i: (i, 0)),
        pl.BlockSpec((BR, D), lambda i: (i, 0)),
        pl.BlockSpec((1, D), lambda i: (0, 0)),
    ],
    out_specs=pl.BlockSpec((BR, D), lambda i: (i, 0)),
    out_shape=jax.ShapeDtypeStruct((N, D), jnp.float32),
)


def kernel(x, edge_index, W1, b1, W2, b2):
    src = edge_index[0]
    dst = edge_index[1]
    cntp = _cnt_call(dst).reshape(2, NP, D)
    dinvb, hs1 = _pre_call(cntp, x, W1)
    p = _agg_call(hs1, src, dst).reshape(2, NP, D)
    hs2 = _mid_call(p, hs1, dinvb, b1.reshape(1, D), W2)
    q = _agg_call(hs2, src, dst).reshape(2, NP, D)
    return _fin_call(q, hs2, dinvb, b2.reshape(1, D))
